# baseline (speedup 1.0000x reference)
import functools

import jax
import jax.numpy as jnp
from jax import lax
from jax.experimental import pallas as pl
from jax.experimental.pallas import tpu as pltpu
from jax.experimental.pallas import tpu_sc as plsc

_D = 128
_NC = 2
_NS = 16
_NW = _NC * _NS
_CH = 128


_K = 2


@functools.partial(jax.jit, static_argnums=0)
def _gather(B, idx2d, tbl):
    n_ch = B // (_NW * _CH)
    n_it = n_ch // _K
    step_rows = _K * _CH

    def body(idx_hbm, table_hbm, out_hbm, idx_v, rows_v, gsem):
        wid = lax.axis_index("s") * _NC + lax.axis_index("c")
        row0 = wid * n_ch
        pltpu.sync_copy(idx_hbm.at[pl.ds(row0, n_ch), :], idx_v)

        def launch(it, b):
            for j in range(_K):
                pltpu.async_copy(
                    table_hbm.at[idx_v.at[it * _K + j]],
                    rows_v.at[b, pl.ds(j * _CH, _CH)],
                    gsem,
                )

        def drain(it, b):
            for j in range(_K):
                pltpu.make_async_copy(
                    table_hbm.at[idx_v.at[it * _K + j]],
                    rows_v.at[b, pl.ds(j * _CH, _CH)],
                    gsem,
                ).wait()

        launch(0, 0)

        def step(it, carry):
            b = lax.rem(it, 2)
            drain(it, b)
            launch(jnp.minimum(it + 1, n_it - 1), 1 - b)
            pltpu.sync_copy(
                rows_v.at[b],
                out_hbm.at[pl.ds((wid * n_it + it) * step_rows, step_rows), :],
            )
            return carry

        lax.fori_loop(0, n_it, step, 0)
        drain(n_it - 1, lax.rem(n_it, 2))

    mesh = plsc.VectorSubcoreMesh(core_axis_name="c", subcore_axis_name="s")
    f = pl.kernel(
        body,
        out_type=jax.ShapeDtypeStruct((B, _D), jnp.float32),
        mesh=mesh,
        scratch_types=[
            pltpu.VMEM((n_ch, _CH), jnp.int32),
            pltpu.VMEM((2, _K * _CH, _D), jnp.float32),
            pltpu.SemaphoreType.DMA,
        ],
    )
    return f(idx2d, tbl)


def kernel(Z, table):
    n, m = Z.shape
    B = n * m
    idx2d = Z.reshape(B // _CH, _CH).astype(jnp.int32)
    tbl = table.at[0].set(0.0)
    out = _gather(B, idx2d, tbl)
    return out.reshape(n, m, _D)


# speedup vs baseline: 1.6179x; 1.6179x over previous
import functools

import jax
import jax.numpy as jnp
from jax import lax
from jax.experimental import pallas as pl
from jax.experimental.pallas import tpu as pltpu
from jax.experimental.pallas import tpu_sc as plsc

_D = 128
_T = 128
_NC = 2
_NS = 16
_NW = _NC * _NS
_CH = 128


@functools.partial(jax.jit, static_argnums=0)
def _gather(B, idx2d, tbl):
    n_ch = B // (_NW * _CH)

    def body(idx_hbm, table_hbm, out_hbm, idx_v, table_v, rows_v, osem):
        wid = lax.axis_index("s") * _NC + lax.axis_index("c")
        row0 = wid * n_ch
        pltpu.sync_copy(table_hbm, table_v)
        pltpu.sync_copy(idx_hbm.at[pl.ds(row0, n_ch), :], idx_v)

        def out_slice(it):
            return out_hbm.at[pl.ds((row0 + it) * _CH, _CH), :]

        def step(it, carry):
            b = lax.rem(it, 2)

            @pl.when(it >= 2)
            def _():
                pltpu.make_async_copy(rows_v.at[b], out_slice(it - 2), osem).wait()

            def group(g, c):
                zv = idx_v[it, pl.ds(g * 16, 16)]
                for k in range(16):
                    z = zv[k]
                    i = g * 16 + k
                    for j in range(_D // 16):
                        rows_v[b, i, pl.ds(j * 16, 16)] = table_v[z, pl.ds(j * 16, 16)]
                return c

            lax.fori_loop(0, _CH // 16, group, 0)
            pltpu.async_copy(rows_v.at[b], out_slice(it), osem)
            return carry

        lax.fori_loop(0, n_ch, step, 0)
        for k in (2, 1):
            it = n_ch - k
            pltpu.make_async_copy(
                rows_v.at[lax.rem(it, 2)], out_slice(it), osem
            ).wait()

    mesh = plsc.VectorSubcoreMesh(core_axis_name="c", subcore_axis_name="s")
    f = pl.kernel(
        body,
        out_type=jax.ShapeDtypeStruct((B, _D), jnp.float32),
        mesh=mesh,
        scratch_types=[
            pltpu.VMEM((n_ch, _CH), jnp.int32),
            pltpu.VMEM((_T, _D), jnp.float32),
            pltpu.VMEM((2, _CH, _D), jnp.float32),
            pltpu.SemaphoreType.DMA,
        ],
    )
    return f(idx2d, tbl)


def kernel(Z, table):
    n, m = Z.shape
    B = n * m
    idx2d = Z.reshape(B // _CH, _CH).astype(jnp.int32)
    tbl = table.at[0].set(0.0)
    out = _gather(B, idx2d, tbl)
    return out.reshape(n, m, _D)
